# Initial kernel scaffold; baseline (speedup 1.0000x reference)
import jax
import jax.numpy as jnp
from jax.experimental import pallas as pl


def kernel(target, source, target_feats, source_feats, W0, gamma0, beta0):
    raise NotImplementedError("write your pallas kernel here")


# speedup vs baseline: 20.0864x; 20.0864x over previous
import jax
import jax.numpy as jnp
from jax.experimental import pallas as pl
from jax.experimental.pallas import tpu as pltpu

_NBLK = 256


def _fused_body(tT_ref, src_ref, sf_ref, tf_ref, w0_ref, y_ref, s_ref, ss_ref):
    b = pl.program_id(0)
    j = pl.program_id(1)
    t = tT_ref[0]
    s = src_ref[0]
    sf = sf_ref[0]
    tf = tf_ref[0]
    w0 = w0_ref[...]

    m = s.shape[0]
    nblk = t.shape[1]

    s2 = jnp.sum(s * s, axis=1, keepdims=True)
    t2 = jnp.sum(t * t, axis=0, keepdims=True)
    p = s2 - 2.0 * jax.lax.dot(s, t, preferred_element_type=jnp.float32)

    iota = jax.lax.broadcasted_iota(jnp.int32, (m, nblk), 0)
    recs = []
    ohs = []
    for _ in range(3):
        mv = jnp.min(p, axis=0, keepdims=True)
        eq = p == mv
        am = jnp.min(jnp.where(eq, iota, m), axis=0, keepdims=True)
        oh = iota == am
        d2 = jnp.maximum(mv + t2, 1e-12)
        recs.append(1.0 / (jnp.sqrt(d2) + 1e-8))
        ohs.append(oh)
        p = jnp.where(oh, jnp.float32(1e30), p)

    norm = recs[0] + recs[1] + recs[2]
    zero = jnp.float32(0.0)
    st = (jnp.where(ohs[0], recs[0] / norm, zero)
          + jnp.where(ohs[1], recs[1] / norm, zero)
          + jnp.where(ohs[2], recs[2] / norm, zero))

    interp = jax.lax.dot(sf, st, preferred_element_type=jnp.float32)
    cat = jnp.concatenate([interp, tf], axis=0)
    y = jax.lax.dot(w0, cat, preferred_element_type=jnp.float32)
    y_ref[0] = y

    @pl.when((b == 0) & (j == 0))
    def _():
        s_ref[...] = jnp.zeros_like(s_ref)
        ss_ref[...] = jnp.zeros_like(ss_ref)

    s_ref[...] += jnp.sum(y, axis=1, keepdims=True)
    ss_ref[...] += jnp.sum(y * y, axis=1, keepdims=True)


def _norm_body(y_ref, sc_ref, sh_ref, o_ref):
    y = y_ref[0]
    o_ref[0] = jnp.maximum(y * sc_ref[...] + sh_ref[...], 0.0)


def kernel(target, source, target_feats, source_feats, W0, gamma0, beta0):
    B, n, _ = target.shape
    m = source.shape[1]
    nb = n // _NBLK
    tT = jnp.transpose(target, (0, 2, 1))

    y_raw, ssum, ssq = pl.pallas_call(
        _fused_body,
        grid=(B, nb),
        in_specs=[
            pl.BlockSpec((1, 3, _NBLK), lambda b, j: (b, 0, j)),
            pl.BlockSpec((1, m, 3), lambda b, j: (b, 0, 0)),
            pl.BlockSpec((1, source_feats.shape[1], m), lambda b, j: (b, 0, 0)),
            pl.BlockSpec((1, target_feats.shape[1], _NBLK), lambda b, j: (b, 0, j)),
            pl.BlockSpec((64, 128), lambda b, j: (0, 0)),
        ],
        out_specs=[
            pl.BlockSpec((1, 64, _NBLK), lambda b, j: (b, 0, j)),
            pl.BlockSpec((64, 1), lambda b, j: (0, 0)),
            pl.BlockSpec((64, 1), lambda b, j: (0, 0)),
        ],
        out_shape=[
            jax.ShapeDtypeStruct((B, 64, n), jnp.float32),
            jax.ShapeDtypeStruct((64, 1), jnp.float32),
            jax.ShapeDtypeStruct((64, 1), jnp.float32),
        ],
        compiler_params=pltpu.CompilerParams(
            dimension_semantics=("arbitrary", "arbitrary")),
    )(tT, source, source_feats, target_feats, W0)

    cnt = jnp.float32(B * n)
    mean = ssum[:, 0] / cnt
    var = ssq[:, 0] / cnt - mean * mean
    scale = gamma0 / jnp.sqrt(var + 1e-5)
    shift = beta0 - mean * scale

    out = pl.pallas_call(
        _norm_body,
        grid=(B, nb),
        in_specs=[
            pl.BlockSpec((1, 64, _NBLK), lambda b, j: (b, 0, j)),
            pl.BlockSpec((64, 1), lambda b, j: (0, 0)),
            pl.BlockSpec((64, 1), lambda b, j: (0, 0)),
        ],
        out_specs=pl.BlockSpec((1, 64, _NBLK), lambda b, j: (b, 0, j)),
        out_shape=jax.ShapeDtypeStruct((B, 64, n), jnp.float32),
        compiler_params=pltpu.CompilerParams(
            dimension_semantics=("parallel", "parallel")),
    )(y_raw, scale.reshape(64, 1), shift.reshape(64, 1))
    return out


# speedup vs baseline: 28.3026x; 1.4090x over previous
import jax
import jax.numpy as jnp
from jax.experimental import pallas as pl
from jax.experimental.pallas import tpu as pltpu

_NBLK = 256


def _fused_body(tT_ref, src_ref, sf_ref, tf_ref, w0_ref, y_ref, s_ref, ss_ref):
    b = pl.program_id(0)
    j = pl.program_id(1)
    t = tT_ref[0]
    s4 = src_ref[0]
    sf = sf_ref[0]
    tf = tf_ref[0]
    w0 = w0_ref[...]

    m = s4.shape[0]
    nblk = t.shape[1]

    tt2 = jnp.sum(t * t, axis=0, keepdims=True)
    t4 = jnp.concatenate([t, jnp.ones((1, nblk), jnp.float32)], axis=0)
    p = jax.lax.dot(s4, t4, preferred_element_type=jnp.float32)

    big = jnp.float32(3e38)
    R, G = 8, 8
    bigrow = jnp.full((R, nblk), big, jnp.float32)
    a1 = [bigrow] * G
    a2 = [bigrow] * G
    a3 = [bigrow] * G
    for i in range(m // R):
        g = i % G
        row = p[i * R:(i + 1) * R, :]
        lo1 = jnp.minimum(a1[g], row)
        hi1 = jnp.maximum(a1[g], row)
        lo2 = jnp.minimum(a2[g], hi1)
        hi2 = jnp.maximum(a2[g], hi1)
        lo3 = jnp.minimum(a3[g], hi2)
        a1[g], a2[g], a3[g] = lo1, lo2, lo3

    cand = jnp.concatenate(a1 + a2 + a3, axis=0)
    v1 = jnp.min(cand, axis=0, keepdims=True)
    c2 = jnp.where(cand == v1, big, cand)
    v2 = jnp.min(c2, axis=0, keepdims=True)
    c3 = jnp.where(c2 == v2, big, c2)
    v3 = jnp.min(c3, axis=0, keepdims=True)

    rec1 = 1.0 / (jnp.sqrt(jnp.maximum(v1 + tt2, 1e-12)) + 1e-8)
    rec2 = 1.0 / (jnp.sqrt(jnp.maximum(v2 + tt2, 1e-12)) + 1e-8)
    rec3 = 1.0 / (jnp.sqrt(jnp.maximum(v3 + tt2, 1e-12)) + 1e-8)
    rnorm = 1.0 / (rec1 + rec2 + rec3)

    st = jnp.where(p == v1, rec1,
                   jnp.where(p == v2, rec2,
                             jnp.where(p == v3, rec3, 0.0)))

    interp = jax.lax.dot(sf, st, preferred_element_type=jnp.float32)
    interp = interp * rnorm
    cat = jnp.concatenate([interp, tf], axis=0)
    y = jax.lax.dot(w0, cat, preferred_element_type=jnp.float32)
    y_ref[0] = y

    @pl.when((b == 0) & (j == 0))
    def _():
        s_ref[...] = jnp.zeros_like(s_ref)
        ss_ref[...] = jnp.zeros_like(ss_ref)

    s_ref[...] += jnp.sum(y, axis=1, keepdims=True)
    ss_ref[...] += jnp.sum(y * y, axis=1, keepdims=True)


def _norm_body(y_ref, sc_ref, sh_ref, o_ref):
    y = y_ref[0]
    o_ref[0] = jnp.maximum(y * sc_ref[...] + sh_ref[...], 0.0)


def kernel(target, source, target_feats, source_feats, W0, gamma0, beta0):
    B, n, _ = target.shape
    m = source.shape[1]
    nb = n // _NBLK
    tT = jnp.transpose(target, (0, 2, 1))
    src_aug = jnp.concatenate(
        [source * (-2.0), jnp.sum(source * source, -1, keepdims=True)], axis=-1)

    y_raw, ssum, ssq = pl.pallas_call(
        _fused_body,
        grid=(B, nb),
        in_specs=[
            pl.BlockSpec((1, 3, _NBLK), lambda b, j: (b, 0, j)),
            pl.BlockSpec((1, m, 4), lambda b, j: (b, 0, 0)),
            pl.BlockSpec((1, source_feats.shape[1], m), lambda b, j: (b, 0, 0)),
            pl.BlockSpec((1, target_feats.shape[1], _NBLK), lambda b, j: (b, 0, j)),
            pl.BlockSpec((64, 128), lambda b, j: (0, 0)),
        ],
        out_specs=[
            pl.BlockSpec((1, 64, _NBLK), lambda b, j: (b, 0, j)),
            pl.BlockSpec((64, 1), lambda b, j: (0, 0)),
            pl.BlockSpec((64, 1), lambda b, j: (0, 0)),
        ],
        out_shape=[
            jax.ShapeDtypeStruct((B, 64, n), jnp.float32),
            jax.ShapeDtypeStruct((64, 1), jnp.float32),
            jax.ShapeDtypeStruct((64, 1), jnp.float32),
        ],
        compiler_params=pltpu.CompilerParams(
            dimension_semantics=("arbitrary", "arbitrary")),
    )(tT, src_aug, source_feats, target_feats, W0)

    cnt = jnp.float32(B * n)
    mean = ssum[:, 0] / cnt
    var = ssq[:, 0] / cnt - mean * mean
    scale = gamma0 / jnp.sqrt(var + 1e-5)
    shift = beta0 - mean * scale

    out = pl.pallas_call(
        _norm_body,
        grid=(B, nb),
        in_specs=[
            pl.BlockSpec((1, 64, _NBLK), lambda b, j: (b, 0, j)),
            pl.BlockSpec((64, 1), lambda b, j: (0, 0)),
            pl.BlockSpec((64, 1), lambda b, j: (0, 0)),
        ],
        out_specs=pl.BlockSpec((1, 64, _NBLK), lambda b, j: (b, 0, j)),
        out_shape=jax.ShapeDtypeStruct((B, 64, n), jnp.float32),
        compiler_params=pltpu.CompilerParams(
            dimension_semantics=("parallel", "parallel")),
    )(y_raw, scale.reshape(64, 1), shift.reshape(64, 1))
    return out


# speedup vs baseline: 28.4985x; 1.0069x over previous
import jax
import jax.numpy as jnp
from jax.experimental import pallas as pl
from jax.experimental.pallas import tpu as pltpu

_NBLK = 256


def _fused_body(tT_ref, src_ref, sf_ref, tf_ref, w0_ref, y_ref, s_ref, ss_ref):
    b = pl.program_id(0)
    j = pl.program_id(1)
    t = tT_ref[0]
    s4 = src_ref[0]
    sf = sf_ref[0]
    tf = tf_ref[0]
    w0 = w0_ref[...]

    m = s4.shape[0]
    nblk = t.shape[1]

    tt2 = jnp.sum(t * t, axis=0, keepdims=True)
    p = (jax.lax.dot(s4[:, :3], t, preferred_element_type=jnp.float32)
         + s4[:, 3:])

    big = jnp.float32(3e38)
    R, G = 8, 8
    bigrow = jnp.full((R, nblk), big, jnp.float32)
    a1 = [bigrow] * G
    a2 = [bigrow] * G
    a3 = [bigrow] * G
    for i in range(m // R):
        g = i % G
        row = p[i * R:(i + 1) * R, :]
        lo1 = jnp.minimum(a1[g], row)
        hi1 = jnp.maximum(a1[g], row)
        lo2 = jnp.minimum(a2[g], hi1)
        hi2 = jnp.maximum(a2[g], hi1)
        lo3 = jnp.minimum(a3[g], hi2)
        a1[g], a2[g], a3[g] = lo1, lo2, lo3

    cand = jnp.concatenate(a1 + a2 + a3, axis=0)
    v1 = jnp.min(cand, axis=0, keepdims=True)
    c2 = jnp.where(cand == v1, big, cand)
    v2 = jnp.min(c2, axis=0, keepdims=True)
    c3 = jnp.where(c2 == v2, big, c2)
    v3 = jnp.min(c3, axis=0, keepdims=True)

    rec1 = 1.0 / (jnp.sqrt(jnp.maximum(v1 + tt2, 1e-12)) + 1e-8)
    rec2 = 1.0 / (jnp.sqrt(jnp.maximum(v2 + tt2, 1e-12)) + 1e-8)
    rec3 = 1.0 / (jnp.sqrt(jnp.maximum(v3 + tt2, 1e-12)) + 1e-8)
    rnorm = 1.0 / (rec1 + rec2 + rec3)

    st = jnp.where(p == v1, rec1,
                   jnp.where(p == v2, rec2,
                             jnp.where(p == v3, rec3, 0.0)))

    interp = jax.lax.dot(sf, st, preferred_element_type=jnp.float32)
    interp = interp * rnorm
    cat = jnp.concatenate([interp, tf], axis=0)
    y = jax.lax.dot(w0, cat, preferred_element_type=jnp.float32)
    y_ref[0] = y

    @pl.when((b == 0) & (j == 0))
    def _():
        s_ref[...] = jnp.zeros_like(s_ref)
        ss_ref[...] = jnp.zeros_like(ss_ref)

    s_ref[...] += jnp.sum(y, axis=1, keepdims=True)
    ss_ref[...] += jnp.sum(y * y, axis=1, keepdims=True)


def _norm_body(y_ref, sc_ref, sh_ref, o_ref):
    y = y_ref[0]
    o_ref[0] = jnp.maximum(y * sc_ref[...] + sh_ref[...], 0.0)


def kernel(target, source, target_feats, source_feats, W0, gamma0, beta0):
    B, n, _ = target.shape
    m = source.shape[1]
    nb = n // _NBLK
    tT = jnp.transpose(target, (0, 2, 1))
    src_aug = jnp.concatenate(
        [source * (-2.0), jnp.sum(source * source, -1, keepdims=True)], axis=-1)

    y_raw, ssum, ssq = pl.pallas_call(
        _fused_body,
        grid=(B, nb),
        in_specs=[
            pl.BlockSpec((1, 3, _NBLK), lambda b, j: (b, 0, j)),
            pl.BlockSpec((1, m, 4), lambda b, j: (b, 0, 0)),
            pl.BlockSpec((1, source_feats.shape[1], m), lambda b, j: (b, 0, 0)),
            pl.BlockSpec((1, target_feats.shape[1], _NBLK), lambda b, j: (b, 0, j)),
            pl.BlockSpec((64, 128), lambda b, j: (0, 0)),
        ],
        out_specs=[
            pl.BlockSpec((1, 64, _NBLK), lambda b, j: (b, 0, j)),
            pl.BlockSpec((64, 1), lambda b, j: (0, 0)),
            pl.BlockSpec((64, 1), lambda b, j: (0, 0)),
        ],
        out_shape=[
            jax.ShapeDtypeStruct((B, 64, n), jnp.float32),
            jax.ShapeDtypeStruct((64, 1), jnp.float32),
            jax.ShapeDtypeStruct((64, 1), jnp.float32),
        ],
        compiler_params=pltpu.CompilerParams(
            dimension_semantics=("arbitrary", "arbitrary")),
    )(tT, src_aug, source_feats, target_feats, W0)

    cnt = jnp.float32(B * n)
    mean = ssum[:, 0] / cnt
    var = ssq[:, 0] / cnt - mean * mean
    scale = gamma0 / jnp.sqrt(var + 1e-5)
    shift = beta0 - mean * scale

    out = pl.pallas_call(
        _norm_body,
        grid=(B, nb),
        in_specs=[
            pl.BlockSpec((1, 64, _NBLK), lambda b, j: (b, 0, j)),
            pl.BlockSpec((64, 1), lambda b, j: (0, 0)),
            pl.BlockSpec((64, 1), lambda b, j: (0, 0)),
        ],
        out_specs=pl.BlockSpec((1, 64, _NBLK), lambda b, j: (b, 0, j)),
        out_shape=jax.ShapeDtypeStruct((B, 64, n), jnp.float32),
        compiler_params=pltpu.CompilerParams(
            dimension_semantics=("parallel", "parallel")),
    )(y_raw, scale.reshape(64, 1), shift.reshape(64, 1))
    return out
